# baseline (device time: 39891 ns/iter reference)
import functools
import os

import jax
import jax.numpy as jnp
from jax import lax
from jax.experimental import pallas as pl
from jax.experimental.pallas import tpu as pltpu

N_DEV = 4
B = 64
D = 512

N_EXCH = int(os.environ.get("KERNEL_EXCH", "6"))

_PREC = {
    "default": jax.lax.Precision.DEFAULT,
    "high": jax.lax.Precision.HIGH,
    "highest": jax.lax.Precision.HIGHEST,
}[os.environ.get("KERNEL_PREC", "highest")]

MODE = os.environ.get("KERNEL_MODE", "full")


def kernel(x, Win0, Wout0, Win1, Wout1, Win2, Wout2):
    if MODE == "empty":
        def empty_body(x_ref, *refs):
            out_ref = refs[6]
            my = lax.axis_index("i")
            out_ref[:, :] = x_ref[pl.ds(my * (B // N_DEV), B // N_DEV), :]

        return pl.pallas_call(
            empty_body,
            out_shape=jax.ShapeDtypeStruct((B // N_DEV, D), jnp.float32),
            in_specs=[pl.BlockSpec(memory_space=pltpu.VMEM)]
            + [pl.BlockSpec(memory_space=pltpu.MemorySpace.HBM)] * 6,
            out_specs=pl.BlockSpec(memory_space=pltpu.VMEM),
        )(x, Win0, Wout0, Win1, Wout1, Win2, Wout2)

    def body(
        x_ref,
        win0_ref,
        wout0_ref,
        win1_ref,
        wout1_ref,
        win2_ref,
        wout2_ref,
        out_ref,
        send_buf,
        comm_ref,
        stage_ref,
        send_sems,
        recv_sems,
    ):
        my = lax.axis_index("i")
        p1 = my ^ 1
        p2 = 3 - my

        if N_EXCH > 0:
            barrier_sem = pltpu.get_barrier_semaphore()
            for p in (p1, p2):
                pl.semaphore_signal(
                    barrier_sem, inc=1, device_id=(p,),
                    device_id_type=pl.DeviceIdType.MESH,
                )

        wins = [win0_ref, win1_ref, win2_ref]
        wouts = [wout0_ref, wout1_ref, wout2_ref]

        pending_sends = []
        xcur = x_ref[:, :]
        for layer in range(3):
            h = jnp.maximum(
                jnp.dot(xcur, wins[layer][:, :], precision=_PREC,
                        preferred_element_type=jnp.float32),
                0.0,
            )
            part = jnp.dot(h, wouts[layer][:, :], precision=_PREC,
                           preferred_element_type=jnp.float32)
            if layer == 0 and N_EXCH > 0:
                pl.semaphore_wait(barrier_sem, 2)
            for stage, partner in enumerate((p1, p2)):
                e = 2 * layer + stage
                if e >= N_EXCH:
                    continue
                send_buf[e, :, :] = part
                rdma = pltpu.make_async_remote_copy(
                    src_ref=send_buf.at[e],
                    dst_ref=comm_ref.at[e],
                    send_sem=send_sems.at[e],
                    recv_sem=recv_sems.at[e],
                    device_id=(partner,),
                    device_id_type=pl.DeviceIdType.MESH,
                )
                rdma.start()
                pending_sends.append(rdma)
                rdma.wait_recv()
                part = part + comm_ref[e, :, :]
            xcur = part

        stage_ref[:, :] = xcur
        out_ref[:, :] = stage_ref[pl.ds(my * (B // N_DEV), B // N_DEV), :]

        for rdma in pending_sends:
            rdma.wait_send()


    return pl.pallas_call(
        body,
        out_shape=jax.ShapeDtypeStruct((B // N_DEV, D), jnp.float32),
        in_specs=[pl.BlockSpec(memory_space=pltpu.VMEM)] * 7,
        out_specs=pl.BlockSpec(memory_space=pltpu.VMEM),
        scratch_shapes=[
            pltpu.VMEM((6, B, D), jnp.float32),
            pltpu.VMEM((6, B, D), jnp.float32),
            pltpu.VMEM((B, D), jnp.float32),
            pltpu.SemaphoreType.DMA((6,)),
            pltpu.SemaphoreType.DMA((6,)),
        ],
        compiler_params=(
            pltpu.CompilerParams(collective_id=0)
            if N_EXCH > 0
            else pltpu.CompilerParams()
        ),
    )(x, Win0, Wout0, Win1, Wout1, Win2, Wout2)


# device time: 33497 ns/iter; 1.1909x vs baseline; 1.1909x over previous
import functools
import os

import jax
import jax.numpy as jnp
from jax import lax
from jax.experimental import pallas as pl
from jax.experimental.pallas import tpu as pltpu

N_DEV = 4
B = 64
D = 512

N_EXCH = int(os.environ.get("KERNEL_EXCH", "6"))

_PREC = {
    "default": jax.lax.Precision.DEFAULT,
    "high": jax.lax.Precision.HIGH,
    "highest": jax.lax.Precision.HIGHEST,
}[os.environ.get("KERNEL_PREC", "default")]

MODE = os.environ.get("KERNEL_MODE", "full")


def kernel(x, Win0, Wout0, Win1, Wout1, Win2, Wout2):
    if MODE == "empty":
        def empty_body(x_ref, *refs):
            out_ref = refs[6]
            my = lax.axis_index("i")
            out_ref[:, :] = x_ref[pl.ds(my * (B // N_DEV), B // N_DEV), :]

        return pl.pallas_call(
            empty_body,
            out_shape=jax.ShapeDtypeStruct((B // N_DEV, D), jnp.float32),
            in_specs=[pl.BlockSpec(memory_space=pltpu.VMEM)]
            + [pl.BlockSpec(memory_space=pltpu.MemorySpace.HBM)] * 6,
            out_specs=pl.BlockSpec(memory_space=pltpu.VMEM),
        )(x, Win0, Wout0, Win1, Wout1, Win2, Wout2)

    def body(
        x_ref,
        win0_ref,
        wout0_ref,
        win1_ref,
        wout1_ref,
        win2_ref,
        wout2_ref,
        out_ref,
        send_buf,
        comm_ref,
        stage_ref,
        send_sems,
        recv_sems,
    ):
        my = lax.axis_index("i")
        p1 = my ^ 1
        p2 = 3 - my

        if N_EXCH > 0:
            barrier_sem = pltpu.get_barrier_semaphore()
            for p in (p1, p2):
                pl.semaphore_signal(
                    barrier_sem, inc=1, device_id=(p,),
                    device_id_type=pl.DeviceIdType.MESH,
                )

        wins = [win0_ref, win1_ref, win2_ref]
        wouts = [wout0_ref, wout1_ref, wout2_ref]

        pending_sends = []
        xcur = x_ref[:, :]
        for layer in range(3):
            h = jnp.maximum(
                jnp.dot(xcur, wins[layer][:, :], precision=_PREC,
                        preferred_element_type=jnp.float32),
                0.0,
            )
            part = jnp.dot(h, wouts[layer][:, :], precision=_PREC,
                           preferred_element_type=jnp.float32)
            if layer == 0 and N_EXCH > 0:
                pl.semaphore_wait(barrier_sem, 2)
            for stage, partner in enumerate((p1, p2)):
                e = 2 * layer + stage
                if e >= N_EXCH:
                    continue
                send_buf[e, :, :] = part
                rdma = pltpu.make_async_remote_copy(
                    src_ref=send_buf.at[e],
                    dst_ref=comm_ref.at[e],
                    send_sem=send_sems.at[e],
                    recv_sem=recv_sems.at[e],
                    device_id=(partner,),
                    device_id_type=pl.DeviceIdType.MESH,
                )
                rdma.start()
                pending_sends.append(rdma)
                rdma.wait_recv()
                part = part + comm_ref[e, :, :]
            xcur = part

        stage_ref[:, :] = xcur
        out_ref[:, :] = stage_ref[pl.ds(my * (B // N_DEV), B // N_DEV), :]

        for rdma in pending_sends:
            rdma.wait_send()


    return pl.pallas_call(
        body,
        out_shape=jax.ShapeDtypeStruct((B // N_DEV, D), jnp.float32),
        in_specs=[pl.BlockSpec(memory_space=pltpu.VMEM)] * 7,
        out_specs=pl.BlockSpec(memory_space=pltpu.VMEM),
        scratch_shapes=[
            pltpu.VMEM((6, B, D), jnp.float32),
            pltpu.VMEM((6, B, D), jnp.float32),
            pltpu.VMEM((B, D), jnp.float32),
            pltpu.SemaphoreType.DMA((6,)),
            pltpu.SemaphoreType.DMA((6,)),
        ],
        compiler_params=(
            pltpu.CompilerParams(collective_id=0)
            if N_EXCH > 0
            else pltpu.CompilerParams()
        ),
    )(x, Win0, Wout0, Win1, Wout1, Win2, Wout2)


# device time: 30009 ns/iter; 1.3293x vs baseline; 1.1162x over previous
import functools
import os

import jax
import jax.numpy as jnp
from jax import lax
from jax.experimental import pallas as pl
from jax.experimental.pallas import tpu as pltpu

N_DEV = 4
B = 64
D = 512
HALF = B // 2

_PREC = {
    "default": jax.lax.Precision.DEFAULT,
    "high": jax.lax.Precision.HIGH,
    "highest": jax.lax.Precision.HIGHEST,
}[os.environ.get("KERNEL_PREC", "default")]


def kernel(x, Win0, Wout0, Win1, Wout1, Win2, Wout2):
    def body(
        x_ref,
        win0_ref,
        wout0_ref,
        win1_ref,
        wout1_ref,
        win2_ref,
        wout2_ref,
        out_ref,
        send_buf,
        comm_ref,
        stage_ref,
        send_sems,
        recv_sems,
    ):
        my = lax.axis_index("i")
        p1 = my ^ 1
        p2 = 3 - my

        barrier_sem = pltpu.get_barrier_semaphore()
        for p in (p1, p2):
            pl.semaphore_signal(
                barrier_sem, inc=1, device_id=(p,),
                device_id_type=pl.DeviceIdType.MESH,
            )

        wins = [win0_ref, win1_ref, win2_ref]
        wouts = [wout0_ref, wout1_ref, wout2_ref]
        pending_sends = []

        def mlp(xh, layer):
            h = jnp.maximum(
                jnp.dot(xh, wins[layer][:, :], precision=_PREC,
                        preferred_element_type=jnp.float32),
                0.0,
            )
            return jnp.dot(h, wouts[layer][:, :], precision=_PREC,
                           preferred_element_type=jnp.float32)

        def exch_start(val, e, peer):
            send_buf[e, :, :] = val
            rdma = pltpu.make_async_remote_copy(
                src_ref=send_buf.at[e],
                dst_ref=comm_ref.at[e],
                send_sem=send_sems.at[e],
                recv_sem=recv_sems.at[e],
                device_id=(peer,),
                device_id_type=pl.DeviceIdType.MESH,
            )
            rdma.start()
            pending_sends.append(rdma)
            return rdma

        def exch_finish(rdma, e):
            rdma.wait_recv()
            return comm_ref[e, :, :]

        xA = x_ref[:HALF, :]
        xB = x_ref[HALF:, :]
        pending_B = None

        for layer in range(3):
            eA0, eB0, eA1, eB1 = (4 * layer + k for k in range(4))

            partA = mlp(xA, layer)
            if layer == 0:
                pl.semaphore_wait(barrier_sem, 2)
            rA0 = exch_start(partA, eA0, p1)

            if pending_B is not None:
                rdma_b, slot_b, part_b = pending_B
                xB = part_b + exch_finish(rdma_b, slot_b)
            partB = mlp(xB, layer)
            rB0 = exch_start(partB, eB0, p1)

            partA = partA + exch_finish(rA0, eA0)
            rA1 = exch_start(partA, eA1, p2)

            partB = partB + exch_finish(rB0, eB0)
            rB1 = exch_start(partB, eB1, p2)

            xA = partA + exch_finish(rA1, eA1)
            pending_B = (rB1, eB1, partB)

        rdma_b, slot_b, part_b = pending_B
        xB = part_b + exch_finish(rdma_b, slot_b)

        stage_ref[:HALF, :] = xA
        stage_ref[HALF:, :] = xB
        out_ref[:, :] = stage_ref[pl.ds(my * (B // N_DEV), B // N_DEV), :]

        for rdma in pending_sends:
            rdma.wait_send()

    return pl.pallas_call(
        body,
        out_shape=jax.ShapeDtypeStruct((B // N_DEV, D), jnp.float32),
        in_specs=[pl.BlockSpec(memory_space=pltpu.VMEM)] * 7,
        out_specs=pl.BlockSpec(memory_space=pltpu.VMEM),
        scratch_shapes=[
            pltpu.VMEM((12, HALF, D), jnp.float32),
            pltpu.VMEM((12, HALF, D), jnp.float32),
            pltpu.VMEM((B, D), jnp.float32),
            pltpu.SemaphoreType.DMA((12,)),
            pltpu.SemaphoreType.DMA((12,)),
        ],
        compiler_params=pltpu.CompilerParams(collective_id=0),
    )(x, Win0, Wout0, Win1, Wout1, Win2, Wout2)


# device time: 27668 ns/iter; 1.4418x vs baseline; 1.0846x over previous
import functools
import os

import jax
import jax.numpy as jnp
from jax import lax
from jax.experimental import pallas as pl
from jax.experimental.pallas import tpu as pltpu

N_DEV = 4
B = 64
D = 512
HALF = B // 2

_PREC = {
    "default": jax.lax.Precision.DEFAULT,
    "high": jax.lax.Precision.HIGH,
    "highest": jax.lax.Precision.HIGHEST,
}[os.environ.get("KERNEL_PREC", "default")]

_WIRE = {"f32": jnp.float32, "bf16": jnp.bfloat16}[
    os.environ.get("KERNEL_WIRE", "bf16")
]


def kernel(x, Win0, Wout0, Win1, Wout1, Win2, Wout2):
    def body(
        x_ref,
        win0_ref,
        wout0_ref,
        win1_ref,
        wout1_ref,
        win2_ref,
        wout2_ref,
        out_ref,
        send_buf,
        comm_ref,
        stage_ref,
        send_sems,
        recv_sems,
    ):
        my = lax.axis_index("i")
        p1 = my ^ 1
        p2 = 3 - my

        barrier_sem = pltpu.get_barrier_semaphore()
        for p in (p1, p2):
            pl.semaphore_signal(
                barrier_sem, inc=1, device_id=(p,),
                device_id_type=pl.DeviceIdType.MESH,
            )

        wins = [win0_ref, win1_ref, win2_ref]
        wouts = [wout0_ref, wout1_ref, wout2_ref]
        pending_sends = []

        def mlp(xh, layer):
            h = jnp.maximum(
                jnp.dot(xh, wins[layer][:, :], precision=_PREC,
                        preferred_element_type=jnp.float32),
                0.0,
            )
            return jnp.dot(h, wouts[layer][:, :], precision=_PREC,
                           preferred_element_type=jnp.float32)

        def exch_start(val, e, peer):
            send_buf[e, :, :] = val.astype(send_buf.dtype)
            rdma = pltpu.make_async_remote_copy(
                src_ref=send_buf.at[e],
                dst_ref=comm_ref.at[e],
                send_sem=send_sems.at[e],
                recv_sem=recv_sems.at[e],
                device_id=(peer,),
                device_id_type=pl.DeviceIdType.MESH,
            )
            rdma.start()
            pending_sends.append(rdma)
            return rdma

        def exch_finish(rdma, e):
            rdma.wait_recv()
            return comm_ref[e, :, :].astype(jnp.float32)

        xA = x_ref[:HALF, :]
        xB = x_ref[HALF:, :]
        pending_B = None

        for layer in range(3):
            eA0, eB0, eA1, eB1 = (4 * layer + k for k in range(4))

            partA = mlp(xA, layer)
            if layer == 0:
                pl.semaphore_wait(barrier_sem, 2)
            rA0 = exch_start(partA, eA0, p1)

            if pending_B is not None:
                rdma_b, slot_b, part_b = pending_B
                xB = part_b + exch_finish(rdma_b, slot_b)
            partB = mlp(xB, layer)
            rB0 = exch_start(partB, eB0, p1)

            partA = partA + exch_finish(rA0, eA0)
            rA1 = exch_start(partA, eA1, p2)

            partB = partB + exch_finish(rB0, eB0)
            rB1 = exch_start(partB, eB1, p2)

            xA = partA + exch_finish(rA1, eA1)
            pending_B = (rB1, eB1, partB)

        rdma_b, slot_b, part_b = pending_B
        xB = part_b + exch_finish(rdma_b, slot_b)

        stage_ref[:HALF, :] = xA
        stage_ref[HALF:, :] = xB
        out_ref[:, :] = stage_ref[pl.ds(my * (B // N_DEV), B // N_DEV), :]

        for rdma in pending_sends:
            rdma.wait_send()

    return pl.pallas_call(
        body,
        out_shape=jax.ShapeDtypeStruct((B // N_DEV, D), jnp.float32),
        in_specs=[pl.BlockSpec(memory_space=pltpu.VMEM)] * 7,
        out_specs=pl.BlockSpec(memory_space=pltpu.VMEM),
        scratch_shapes=[
            pltpu.VMEM((12, HALF, D), _WIRE),
            pltpu.VMEM((12, HALF, D), _WIRE),
            pltpu.VMEM((B, D), jnp.float32),
            pltpu.SemaphoreType.DMA((12,)),
            pltpu.SemaphoreType.DMA((12,)),
        ],
        compiler_params=pltpu.CompilerParams(collective_id=0),
    )(x, Win0, Wout0, Win1, Wout1, Win2, Wout2)


# device time: 24459 ns/iter; 1.6309x vs baseline; 1.1312x over previous
import functools
import os

import jax
import jax.numpy as jnp
from jax import lax
from jax.experimental import pallas as pl
from jax.experimental.pallas import tpu as pltpu

N_DEV = 4
B = 64
D = 512
HALF = B // 2

_PREC = {
    "default": jax.lax.Precision.DEFAULT,
    "high": jax.lax.Precision.HIGH,
    "highest": jax.lax.Precision.HIGHEST,
}[os.environ.get("KERNEL_PREC", "default")]

_WIRE = {"f32": jnp.float32, "bf16": jnp.bfloat16}[
    os.environ.get("KERNEL_WIRE", "bf16")
]


def kernel(x, Win0, Wout0, Win1, Wout1, Win2, Wout2):
    def body(
        x_ref,
        win0_ref,
        wout0_ref,
        win1_ref,
        wout1_ref,
        win2_ref,
        wout2_ref,
        out_ref,
        send_buf,
        comm_ref,
        stage_ref,
        send_sems,
        recv_sems,
    ):
        my = lax.axis_index("i")
        peers = (my ^ 1, 3 - my, my ^ 2)

        barrier_sem = pltpu.get_barrier_semaphore()
        for p in peers:
            pl.semaphore_signal(
                barrier_sem, inc=1, device_id=(p,),
                device_id_type=pl.DeviceIdType.MESH,
            )

        wins = [win0_ref, win1_ref, win2_ref]
        wouts = [wout0_ref, wout1_ref, wout2_ref]
        pending_sends = []

        def mlp(xh, layer):
            h = jnp.maximum(
                jnp.dot(xh, wins[layer][:, :], precision=_PREC,
                        preferred_element_type=jnp.float32),
                0.0,
            )
            return jnp.dot(h, wouts[layer][:, :], precision=_PREC,
                           preferred_element_type=jnp.float32)

        def bcast_start(val, layer, half):
            s = 2 * layer + half
            send_buf[s, :, :] = val.astype(send_buf.dtype)
            rdmas = []
            for k, peer in enumerate(peers):
                e = 6 * layer + 3 * half + k
                rdma = pltpu.make_async_remote_copy(
                    src_ref=send_buf.at[s],
                    dst_ref=comm_ref.at[e],
                    send_sem=send_sems.at[e],
                    recv_sem=recv_sems.at[e],
                    device_id=(peer,),
                    device_id_type=pl.DeviceIdType.MESH,
                )
                rdma.start()
                pending_sends.append(rdma)
                rdmas.append(rdma)
            return rdmas

        def bcast_finish(rdmas, layer, half, own):
            base = 6 * layer + 3 * half
            for rdma in rdmas:
                rdma.wait_recv()
            rsum = (
                comm_ref[base, :, :]
                + comm_ref[base + 1, :, :]
                + comm_ref[base + 2, :, :]
            )
            return own + rsum.astype(jnp.float32)

        xA = x_ref[:HALF, :]
        xB = x_ref[HALF:, :]
        pending_B = None

        for layer in range(3):
            partA = mlp(xA, layer)
            if layer == 0:
                pl.semaphore_wait(barrier_sem, 3)
            rA = bcast_start(partA, layer, 0)

            if pending_B is not None:
                r_b, lyr_b, part_b = pending_B
                xB = bcast_finish(r_b, lyr_b, 1, part_b)
            partB = mlp(xB, layer)
            rB = bcast_start(partB, layer, 1)

            xA = bcast_finish(rA, layer, 0, partA)
            pending_B = (rB, layer, partB)

        r_b, lyr_b, part_b = pending_B
        xB = bcast_finish(r_b, lyr_b, 1, part_b)

        stage_ref[:HALF, :] = xA
        stage_ref[HALF:, :] = xB
        out_ref[:, :] = stage_ref[pl.ds(my * (B // N_DEV), B // N_DEV), :]

        for rdma in pending_sends:
            rdma.wait_send()

    return pl.pallas_call(
        body,
        out_shape=jax.ShapeDtypeStruct((B // N_DEV, D), jnp.float32),
        in_specs=[pl.BlockSpec(memory_space=pltpu.VMEM)] * 7,
        out_specs=pl.BlockSpec(memory_space=pltpu.VMEM),
        scratch_shapes=[
            pltpu.VMEM((6, HALF, D), _WIRE),
            pltpu.VMEM((18, HALF, D), _WIRE),
            pltpu.VMEM((B, D), jnp.float32),
            pltpu.SemaphoreType.DMA((18,)),
            pltpu.SemaphoreType.DMA((18,)),
        ],
        compiler_params=pltpu.CompilerParams(collective_id=0),
    )(x, Win0, Wout0, Win1, Wout1, Win2, Wout2)
